# baseline (speedup 1.0000x reference)
import functools

import jax
import jax.numpy as jnp
from jax import lax
from jax.experimental import pallas as pl
from jax.experimental.pallas import tpu as pltpu
from jax.experimental.pallas import tpu_sc as plsc


def _mm(a, b):
    return jax.lax.dot_general(a, b, (((1,), (0,)), ((), ())),
                               preferred_element_type=jnp.float32,
                               precision=jax.lax.Precision.DEFAULT)


def _prep_kernel(wr1_ref, wr2_ref, wrd2_ref, wrd3_ref, wrd4_ref,
                 m1_ref, m2_ref, n2_ref, n3_ref, n4_ref):
    m1_ref[...] = jnp.zeros((512, 256), jnp.float32)
    w96 = wr1_ref[...]
    for j in range(4):
        m1_ref[128 * j:128 * j + 96, 64 * j:64 * (j + 1)] = w96

    m2_ref[...] = jnp.zeros((448, 256), jnp.float32)
    w192 = wr2_ref[...]
    m2_ref[0:192, 0:128] = w192
    m2_ref[256:448, 128:256] = w192

    n2_ref[...] = jnp.zeros((512, 512), jnp.float32)
    t0 = wrd2_ref[0:256, :]
    t1 = wrd2_ref[256:512, :]
    t2 = wrd2_ref[512:768, :]
    n2_ref[0:256, 0:128] = t1
    n2_ref[0:256, 128:256] = t0
    n2_ref[256:512, 128:256] = t2
    n2_ref[256:512, 256:384] = t1
    n2_ref[256:512, 384:512] = t0

    n3_ref[...] = jnp.zeros((512, 512), jnp.float32)
    u0 = wrd3_ref[0:128, :]
    u1 = wrd3_ref[128:256, :]
    u2 = wrd3_ref[256:384, :]
    for i in range(4):
        n3_ref[128 * i:128 * i + 128, 128 * i:128 * i + 64] = u1
        n3_ref[128 * i:128 * i + 128, 128 * i + 64:128 * i + 128] = u0
        if i < 3:
            n3_ref[128 * (i + 1):128 * (i + 2),
                   128 * i + 64:128 * i + 128] = u2

    n4_ref[...] = jnp.zeros((512, 512), jnp.float32)
    v0 = wrd4_ref[0:64, :]
    v1 = wrd4_ref[64:128, :]
    v2 = wrd4_ref[128:192, :]
    for i in range(8):
        n4_ref[64 * i:64 * i + 64, 64 * i:64 * i + 32] = v1
        n4_ref[64 * i:64 * i + 64, 64 * i + 32:64 * i + 64] = v0
        if i < 7:
            n4_ref[64 * (i + 1):64 * (i + 2), 64 * i + 32:64 * i + 64] = v2


def _shift_helpers(R):
    rid = jax.lax.broadcasted_iota(jnp.int32, (R, 1), 0)
    down_ok = ((rid % 128) != 0).astype(jnp.float32)
    up_ok = ((rid % 128) != 127).astype(jnp.float32)

    def shift_down(v):
        s = jnp.concatenate([jnp.zeros((1, v.shape[1]), v.dtype), v[:-1, :]],
                            axis=0)
        return s * down_ok

    def shift_up(v):
        s = jnp.concatenate([v[1:, :], jnp.zeros((1, v.shape[1]), v.dtype)],
                            axis=0)
        return s * up_ok

    return shift_down, shift_up


def _tile(b, n):
    return jnp.concatenate([b] * n, axis=1)


def _enc_kernel(x_ref, m1_ref, b1_ref, m2_ref, b2_ref,
                m3_ref, b3_ref, wp_ref, pb_ref,
                en_ref, e_ref, ze_ref, idx_ref, *, nb):
    R = nb * 128
    shift_down, _ = _shift_helpers(R)
    x = x_ref[...].reshape(R, 256)
    z32 = jnp.zeros((R, 32), jnp.float32)
    z64 = jnp.zeros((R, 64), jnp.float32)

    p1 = jnp.concatenate([
        shift_down(x[:, 224:256]), x[:, 0:64], z32,
        x[:, 32:128], z32, x[:, 96:192], z32, x[:, 160:256], z32], axis=1)
    h1 = jax.nn.relu(_mm(p1, m1_ref[...]) + _tile(b1_ref[...], 4))

    p2 = jnp.concatenate([
        shift_down(h1[:, 192:256]), h1[:, 0:128], z64, h1[:, 64:256]], axis=1)
    h2 = jax.nn.relu(_mm(p2, m2_ref[...]) + _tile(b2_ref[...], 2))

    p3 = jnp.concatenate([shift_down(h2[:, 128:256]), h2], axis=1)
    h3 = jax.nn.relu(_mm(p3, m3_ref[...]) + b3_ref[...])

    z_e = _mm(h3, wp_ref[...]) + pb_ref[...]
    ze_ref[...] = z_e

    zn = jnp.sum(z_e * z_e, axis=1, keepdims=True)
    prod = jax.lax.dot_general(z_e, e_ref[...], (((1,), (1,)), ((), ())),
                               preferred_element_type=jnp.float32,
                               precision=jax.lax.Precision.DEFAULT)
    dist = zn - 2.0 * prod + en_ref[...]
    mins = jnp.min(dist, axis=1, keepdims=True)
    ii = jax.lax.broadcasted_iota(jnp.int32, (R, 1024), 1)
    cand = jnp.where(dist <= mins, ii, 1024)
    idx_ref[...] = jnp.min(cand, axis=1, keepdims=True)


def _dec_kernel(ze_ref, zq_ref, wrd1_ref, c1_ref, n2_ref, wrd2_ref, c2_ref,
                n3_ref, wrd3_ref, c3_ref, n4_ref, wrd4_ref, c4_ref,
                out_ref, loss_ref, *, nb):
    R = nb * 128
    _, shift_up = _shift_helpers(R)
    z_e = ze_ref[...]
    z_q = zq_ref[...]

    diff = z_e - z_q
    part = jnp.sum(diff * diff, axis=(0, 1), keepdims=True)
    i = pl.program_id(0)

    @pl.when(i == 0)
    def _():
        loss_ref[...] = jnp.zeros((1, 1), jnp.float32)

    loss_ref[...] += part

    a1 = _mm(z_q, wrd1_ref[256:512, :])
    o1 = _mm(z_q, wrd1_ref[0:256, :]) + _mm(shift_up(z_q),
                                            wrd1_ref[512:768, :])
    d1 = jax.nn.relu(jnp.concatenate([a1, o1], axis=1)
                     + _tile(c1_ref[...], 2))

    d2 = _mm(d1, n2_ref[...])
    g2 = _mm(shift_up(d1[:, :256]), wrd2_ref[512:768, :])
    d2 = jnp.concatenate([d2[:, :384], d2[:, 384:] + g2], axis=1)
    d2 = jax.nn.relu(d2 + _tile(c2_ref[...], 4))

    d3 = _mm(d2, n3_ref[...])
    g3 = _mm(shift_up(d2[:, :128]), wrd3_ref[256:384, :])
    d3 = jnp.concatenate([d3[:, :448], d3[:, 448:] + g3], axis=1)
    d3 = jax.nn.relu(d3 + _tile(c3_ref[...], 8))

    d4 = _mm(d3, n4_ref[...])
    g4 = _mm(shift_up(d3[:, :64]), wrd4_ref[128:192, :])
    d4 = jnp.concatenate([d4[:, :480], d4[:, 480:] + g4], axis=1)
    d4 = d4 + _tile(c4_ref[...], 16)

    out_ref[...] = d4.reshape(nb, 128, 512)


def _sc_gather(table, idx):
    info = plsc.get_sparse_core_info()
    nw = info.num_cores * info.num_subcores
    B, D = idx.shape[0], table.shape[1]
    b_per_w = B // nw
    mesh = plsc.VectorSubcoreMesh(core_axis_name="c", subcore_axis_name="s")

    @functools.partial(
        pl.kernel, mesh=mesh,
        out_type=jax.ShapeDtypeStruct((B, D), jnp.float32),
        scratch_types=[
            pltpu.VMEM((b_per_w,), jnp.int32),
            pltpu.VMEM((b_per_w, D), jnp.float32),
            pltpu.SemaphoreType.DMA,
        ],
    )
    def k(table_hbm, idx_hbm, out_hbm, idx_v, rows_v, sem):
        wid = lax.axis_index("s") * info.num_cores + lax.axis_index("c")
        base = wid * b_per_w
        pltpu.sync_copy(idx_hbm.at[pl.ds(base, b_per_w)], idx_v)
        pltpu.async_copy(table_hbm.at[idx_v], rows_v, sem).wait()
        pltpu.sync_copy(rows_v, out_hbm.at[pl.ds(base, b_per_w)])

    return k(table, idx)


def kernel(x, ew1, eb1, ew2, eb2, ew3, eb3, pw, pb,
           dw1, db1, dw2, db2, dw3, db3, dw4, db4, embedding):
    NB = 16
    B = 64
    R = NB * 128

    def taps(w):
        o, i, _ = w.shape
        return jnp.transpose(w, (2, 1, 0)).reshape(3 * i, o)

    wr1 = taps(ew1)
    wr2 = taps(ew2)
    wr3 = taps(ew3)
    wrp = pw[:, :, 0].T
    wrd1 = taps(dw1)
    wrd2 = taps(dw2)
    wrd3 = taps(dw3)
    wrd4 = taps(dw4)
    en = jnp.sum(embedding ** 2, axis=1)[None, :]

    fullb = lambda shp: pl.BlockSpec(shp, lambda: tuple(0 for _ in shp))
    m1, m2, n2, n3, n4 = pl.pallas_call(
        _prep_kernel,
        in_specs=[fullb((96, 64)), fullb((192, 128)), fullb((768, 128)),
                  fullb((384, 64)), fullb((192, 32))],
        out_specs=[fullb((512, 256)), fullb((448, 256)), fullb((512, 512)),
                   fullb((512, 512)), fullb((512, 512))],
        out_shape=[jax.ShapeDtypeStruct(s, jnp.float32) for s in
                   [(512, 256), (448, 256), (512, 512), (512, 512),
                    (512, 512)]],
    )(wr1, wr2, wrd2, wrd3, wrd4)

    x8 = x.reshape(B, 128, 256)
    full = lambda shp: pl.BlockSpec(shp, lambda i: tuple(0 for _ in shp))
    grid = B // NB

    z_e, idx = pl.pallas_call(
        functools.partial(_enc_kernel, nb=NB),
        grid=(grid,),
        in_specs=[
            pl.BlockSpec((NB, 128, 256), lambda i: (i, 0, 0)),
            full((512, 256)), full((1, 64)),
            full((448, 256)), full((1, 128)),
            full((384, 256)), full((1, 256)),
            full((256, 256)), full((1, 256)),
            full((1, 1024)), full((1024, 256)),
        ],
        out_specs=[
            pl.BlockSpec((R, 256), lambda i: (i, 0)),
            pl.BlockSpec((R, 1), lambda i: (i, 0)),
        ],
        out_shape=[
            jax.ShapeDtypeStruct((B * 128, 256), jnp.float32),
            jax.ShapeDtypeStruct((B * 128, 1), jnp.int32),
        ],
    )(x8, m1, eb1[None, :], m2, eb2[None, :], wr3, eb3[None, :],
      wrp, pb[None, :], en, embedding)

    z_q = _sc_gather(embedding, idx.reshape(B * 128))

    out, loss = pl.pallas_call(
        functools.partial(_dec_kernel, nb=NB),
        grid=(grid,),
        in_specs=[
            pl.BlockSpec((R, 256), lambda i: (i, 0)),
            pl.BlockSpec((R, 256), lambda i: (i, 0)),
            full((768, 256)), full((1, 256)),
            full((512, 512)), full((768, 128)), full((1, 128)),
            full((512, 512)), full((384, 64)), full((1, 64)),
            full((512, 512)), full((192, 32)), full((1, 32)),
        ],
        out_specs=[
            pl.BlockSpec((NB, 128, 512), lambda i: (i, 0, 0)),
            pl.BlockSpec((1, 1), lambda i: (0, 0)),
        ],
        out_shape=[
            jax.ShapeDtypeStruct((B, 128, 512), jnp.float32),
            jax.ShapeDtypeStruct((1, 1), jnp.float32),
        ],
    )(z_e, z_q, wrd1, db1[None, :], n2, wrd2, db2[None, :],
      n3, wrd3, db3[None, :], n4, wrd4, db4[None, :])

    out_final = out.reshape(B, 2048, 32)
    indices = idx.reshape(B, 128)
    lossv = loss[0, 0] / (B * 128 * 256)
    return (out_final, lossv, lossv, indices)


# speedup vs baseline: 3.1664x; 3.1664x over previous
import functools

import jax
import jax.numpy as jnp
from jax.experimental import pallas as pl


def _mm(a, b):
    return jax.lax.dot_general(a, b, (((1,), (0,)), ((), ())),
                               preferred_element_type=jnp.float32,
                               precision=jax.lax.Precision.DEFAULT)


def _prep_kernel(wr1_ref, wr2_ref, wrd2_ref, wrd3_ref, wrd4_ref,
                 m1_ref, m2_ref, n2_ref, n3_ref, n4_ref):
    m1_ref[...] = jnp.zeros((512, 256), jnp.float32)
    w96 = wr1_ref[...]
    for j in range(4):
        m1_ref[128 * j:128 * j + 96, 64 * j:64 * (j + 1)] = w96

    m2_ref[...] = jnp.zeros((448, 256), jnp.float32)
    w192 = wr2_ref[...]
    m2_ref[0:192, 0:128] = w192
    m2_ref[256:448, 128:256] = w192

    n2_ref[...] = jnp.zeros((512, 512), jnp.float32)
    t0 = wrd2_ref[0:256, :]
    t1 = wrd2_ref[256:512, :]
    t2 = wrd2_ref[512:768, :]
    n2_ref[0:256, 0:128] = t1
    n2_ref[0:256, 128:256] = t0
    n2_ref[256:512, 128:256] = t2
    n2_ref[256:512, 256:384] = t1
    n2_ref[256:512, 384:512] = t0

    n3_ref[...] = jnp.zeros((512, 512), jnp.float32)
    u0 = wrd3_ref[0:128, :]
    u1 = wrd3_ref[128:256, :]
    u2 = wrd3_ref[256:384, :]
    for i in range(4):
        n3_ref[128 * i:128 * i + 128, 128 * i:128 * i + 64] = u1
        n3_ref[128 * i:128 * i + 128, 128 * i + 64:128 * i + 128] = u0
        if i < 3:
            n3_ref[128 * (i + 1):128 * (i + 2),
                   128 * i + 64:128 * i + 128] = u2

    n4_ref[...] = jnp.zeros((512, 512), jnp.float32)
    v0 = wrd4_ref[0:64, :]
    v1 = wrd4_ref[64:128, :]
    v2 = wrd4_ref[128:192, :]
    for i in range(8):
        n4_ref[64 * i:64 * i + 64, 64 * i:64 * i + 32] = v1
        n4_ref[64 * i:64 * i + 64, 64 * i + 32:64 * i + 64] = v0
        if i < 7:
            n4_ref[64 * (i + 1):64 * (i + 2), 64 * i + 32:64 * i + 64] = v2


def _vq_kernel(x_ref, m1_ref, b1_ref, m2_ref, b2_ref,
               m3_ref, b3_ref, wp_ref, pb_ref,
               en_ref, e_ref,
               wrd1_ref, c1_ref, n2_ref, wrd2_ref, c2_ref,
               n3_ref, wrd3_ref, c3_ref, n4_ref, wrd4_ref, c4_ref,
               out_ref, idx_ref, loss_ref, *, nb):
    R = nb * 128
    rid = jax.lax.broadcasted_iota(jnp.int32, (R, 1), 0)
    down_ok = ((rid % 128) != 0).astype(jnp.float32)
    up_ok = ((rid % 128) != 127).astype(jnp.float32)

    def shift_down(v):
        s = jnp.concatenate([jnp.zeros((1, v.shape[1]), v.dtype), v[:-1, :]],
                            axis=0)
        return s * down_ok

    def shift_up(v):
        s = jnp.concatenate([v[1:, :], jnp.zeros((1, v.shape[1]), v.dtype)],
                            axis=0)
        return s * up_ok

    def tile(b, n):
        return jnp.concatenate([b] * n, axis=1)

    x = x_ref[...].reshape(R, 256)
    z32 = jnp.zeros((R, 32), jnp.float32)
    z64 = jnp.zeros((R, 64), jnp.float32)

    p1 = jnp.concatenate([
        shift_down(x[:, 224:256]), x[:, 0:64], z32,
        x[:, 32:128], z32, x[:, 96:192], z32, x[:, 160:256], z32], axis=1)
    h1 = jax.nn.relu(_mm(p1, m1_ref[...]) + tile(b1_ref[...], 4))

    p2 = jnp.concatenate([
        shift_down(h1[:, 192:256]), h1[:, 0:128], z64, h1[:, 64:256]], axis=1)
    h2 = jax.nn.relu(_mm(p2, m2_ref[...]) + tile(b2_ref[...], 2))

    p3 = jnp.concatenate([shift_down(h2[:, 128:256]), h2], axis=1)
    h3 = jax.nn.relu(_mm(p3, m3_ref[...]) + b3_ref[...])

    z_e = _mm(h3, wp_ref[...]) + pb_ref[...]

    zn = jnp.sum(z_e * z_e, axis=1, keepdims=True)
    prod = jax.lax.dot_general(z_e, e_ref[...], (((1,), (1,)), ((), ())),
                               preferred_element_type=jnp.float32,
                               precision=jax.lax.Precision.DEFAULT)
    dist = zn - 2.0 * prod + en_ref[...]
    mins = jnp.min(dist, axis=1, keepdims=True)
    ii = jax.lax.broadcasted_iota(jnp.int32, (R, 1024), 1)
    cand = jnp.where(dist <= mins, ii, 1024)
    idx = jnp.min(cand, axis=1, keepdims=True)
    idx_ref[...] = idx

    oh = (ii == idx).astype(jnp.float32)
    z_q = _mm(oh, e_ref[...])

    diff = z_e - z_q
    part = jnp.sum(diff * diff, axis=(0, 1), keepdims=True)
    i = pl.program_id(0)

    @pl.when(i == 0)
    def _():
        loss_ref[...] = jnp.zeros((1, 1), jnp.float32)

    loss_ref[...] += part

    a1 = _mm(z_q, wrd1_ref[256:512, :])
    o1 = _mm(z_q, wrd1_ref[0:256, :]) + _mm(shift_up(z_q),
                                            wrd1_ref[512:768, :])
    d1 = jax.nn.relu(jnp.concatenate([a1, o1], axis=1) + tile(c1_ref[...], 2))

    d2 = _mm(d1, n2_ref[...])
    g2 = _mm(shift_up(d1[:, :256]), wrd2_ref[512:768, :])
    d2 = jnp.concatenate([d2[:, :384], d2[:, 384:] + g2], axis=1)
    d2 = jax.nn.relu(d2 + tile(c2_ref[...], 4))

    d3 = _mm(d2, n3_ref[...])
    g3 = _mm(shift_up(d2[:, :128]), wrd3_ref[256:384, :])
    d3 = jnp.concatenate([d3[:, :448], d3[:, 448:] + g3], axis=1)
    d3 = jax.nn.relu(d3 + tile(c3_ref[...], 8))

    d4 = _mm(d3, n4_ref[...])
    g4 = _mm(shift_up(d3[:, :64]), wrd4_ref[128:192, :])
    d4 = jnp.concatenate([d4[:, :480], d4[:, 480:] + g4], axis=1)
    d4 = d4 + tile(c4_ref[...], 16)

    out_ref[...] = d4.reshape(nb, 128, 512)


def kernel(x, ew1, eb1, ew2, eb2, ew3, eb3, pw, pb,
           dw1, db1, dw2, db2, dw3, db3, dw4, db4, embedding):
    NB = 16
    B = 64
    R = NB * 128

    def taps(w):
        o, i, _ = w.shape
        return jnp.transpose(w, (2, 1, 0)).reshape(3 * i, o)

    wr1 = taps(ew1)
    wr2 = taps(ew2)
    wr3 = taps(ew3)
    wrp = pw[:, :, 0].T
    wrd1 = taps(dw1)
    wrd2 = taps(dw2)
    wrd3 = taps(dw3)
    wrd4 = taps(dw4)
    en = jnp.sum(embedding ** 2, axis=1)[None, :]

    fullb = lambda shp: pl.BlockSpec(shp, lambda: tuple(0 for _ in shp))
    m1, m2, n2, n3, n4 = pl.pallas_call(
        _prep_kernel,
        in_specs=[fullb((96, 64)), fullb((192, 128)), fullb((768, 128)),
                  fullb((384, 64)), fullb((192, 32))],
        out_specs=[fullb((512, 256)), fullb((448, 256)), fullb((512, 512)),
                   fullb((512, 512)), fullb((512, 512))],
        out_shape=[jax.ShapeDtypeStruct(s, jnp.float32) for s in
                   [(512, 256), (448, 256), (512, 512), (512, 512),
                    (512, 512)]],
    )(wr1, wr2, wrd2, wrd3, wrd4)

    x8 = x.reshape(B, 128, 256)
    full = lambda shp: pl.BlockSpec(shp, lambda i: tuple(0 for _ in shp))
    grid = B // NB

    out, idx, loss = pl.pallas_call(
        functools.partial(_vq_kernel, nb=NB),
        grid=(grid,),
        in_specs=[
            pl.BlockSpec((NB, 128, 256), lambda i: (i, 0, 0)),
            full((512, 256)), full((1, 64)),
            full((448, 256)), full((1, 128)),
            full((384, 256)), full((1, 256)),
            full((256, 256)), full((1, 256)),
            full((1, 1024)), full((1024, 256)),
            full((768, 256)), full((1, 256)),
            full((512, 512)), full((768, 128)), full((1, 128)),
            full((512, 512)), full((384, 64)), full((1, 64)),
            full((512, 512)), full((192, 32)), full((1, 32)),
        ],
        out_specs=[
            pl.BlockSpec((NB, 128, 512), lambda i: (i, 0, 0)),
            pl.BlockSpec((R, 1), lambda i: (i, 0)),
            pl.BlockSpec((1, 1), lambda i: (0, 0)),
        ],
        out_shape=[
            jax.ShapeDtypeStruct((B, 128, 512), jnp.float32),
            jax.ShapeDtypeStruct((B * 128, 1), jnp.int32),
            jax.ShapeDtypeStruct((1, 1), jnp.float32),
        ],
    )(x8, m1, eb1[None, :], m2, eb2[None, :], wr3, eb3[None, :],
      wrp, pb[None, :], en, embedding,
      wrd1, db1[None, :], n2, wrd2, db2[None, :],
      n3, wrd3, db3[None, :], n4, wrd4, db4[None, :])

    out_final = out.reshape(B, 2048, 32)
    indices = idx.reshape(B, 128)
    lossv = loss[0, 0] / (B * 128 * 256)
    return (out_final, lossv, lossv, indices)
